# baseline (speedup 1.0000x reference)
import functools

import jax
import jax.numpy as jnp
from jax import lax
from jax.experimental import pallas as pl
from jax.experimental.pallas import tpu as pltpu
from jax.experimental.pallas import tpu_sc as plsc

_B, _N, _NREL, _NF = 4, 1024, 8192, 128
_BN, _E = _B * _N, _B * _NREL
_PIN = 14
_SD = 3

_NC, _NS, _L = 2, 16, 16
_NW = _NC * _NS
_EW = _E // _NW
_CH = 128
_NCH = _EW // _CH
_EH = _E // 2
_EWH = _EH // _NW
_NCHH = _EWH // _CH
_RPT = _BN // _NS


_IDXBLK = 1024
_IDXNB = _NREL // _IDXBLK


def _idx_body(rr_ref, rs_ref, gr_ref, gs_ref):
    b = pl.program_id(0)
    iota = lax.broadcasted_iota(jnp.int32, (1, _N), 1).astype(jnp.float32)
    base = b * _N
    gr_ref[0, 0, :] = jnp.sum(rr_ref[0] * iota, axis=1).astype(jnp.int32) + base
    gs_ref[0, 0, :] = jnp.sum(rs_ref[0] * iota, axis=1).astype(jnp.int32) + base


_IDXNBH = _IDXNB // 2


def _make_idx(off):
    nb = _IDXNBH
    in_map = lambda b, j: (b, j + off, 0)
    out_map = lambda b, j: (b * nb + j, 0, 0)
    return pl.pallas_call(
        _idx_body,
        grid=(_B, nb),
        in_specs=[
            pl.BlockSpec((1, _IDXBLK, _N), in_map),
            pl.BlockSpec((1, _IDXBLK, _N), in_map),
        ],
        out_specs=[
            pl.BlockSpec((1, 1, _IDXBLK), out_map),
            pl.BlockSpec((1, 1, _IDXBLK), out_map),
        ],
        out_shape=[
            jax.ShapeDtypeStruct((_B * nb, 1, _IDXBLK), jnp.int32),
            jax.ShapeDtypeStruct((_B * nb, 1, _IDXBLK), jnp.int32),
        ],
    )


_idx_a = _make_idx(0)
_idx_b = _make_idx(_IDXNBH)


def _pre_body(p_in_ref, pe_w0_ref, pe_b0_ref, pe_w1_ref, pe_b1_ref,
              re_w0a_ref, re_w0b_ref, re_b0_ref, rp_w1_ref, rp_w2_ref,
              pp_w0_ref, pp_b_ref,
              enc_p_ref, ar_ref, as_ref, pbias_ref, er_ref, es_ref):
    p_in = p_in_ref[...]
    h = jnp.maximum(jnp.dot(p_in, pe_w0_ref[...]) + pe_b0_ref[...], 0.0)
    enc_p = jnp.maximum(jnp.dot(h, pe_w1_ref[...]) + pe_b1_ref[...], 0.0)
    enc_p_ref[...] = enc_p
    ar_ref[...] = jnp.dot(p_in, re_w0a_ref[...]) + re_b0_ref[...]
    as_ref[...] = jnp.dot(p_in, re_w0b_ref[...])
    pbias_ref[...] = jnp.dot(enc_p, pp_w0_ref[...]) + pp_b_ref[...]
    er_ref[...] = jnp.dot(enc_p, rp_w1_ref[...])
    es_ref[...] = jnp.dot(enc_p, rp_w2_ref[...])


_pre_call = pl.pallas_call(
    _pre_body,
    out_shape=[jax.ShapeDtypeStruct((_BN, _NF), jnp.float32)] * 6,
)


_RBLK = 4096


def _relmlp_body(h0_ref, w1_ref, b1_ref, w2_ref, b2_ref, w3_ref, b3_ref, out_ref):
    h1 = jnp.maximum(jnp.dot(h0_ref[...], w1_ref[...]) + b1_ref[...], 0.0)
    h2 = jnp.maximum(jnp.dot(h1, w2_ref[...]) + b2_ref[...], 0.0)
    out_ref[...] = jnp.dot(h2, w3_ref[...]) + b3_ref[...]


_w_spec = pl.BlockSpec((_NF, _NF), lambda i: (0, 0))
_b_spec = pl.BlockSpec((1, _NF), lambda i: (0, 0))

_relmlp_call = pl.pallas_call(
    _relmlp_body,
    grid=(_EH // _RBLK,),
    in_specs=[pl.BlockSpec((_RBLK, _NF), lambda i: (i, 0)),
              _w_spec, _b_spec, _w_spec, _b_spec, _w_spec, _b_spec],
    out_specs=pl.BlockSpec((_RBLK, _NF), lambda i: (i, 0)),
    out_shape=jax.ShapeDtypeStruct((_EH, _NF), jnp.float32),
)


_sc_mesh = plsc.VectorSubcoreMesh(core_axis_name="c", subcore_axis_name="s")


@functools.partial(
    pl.kernel,
    out_type=jax.ShapeDtypeStruct((_EH, _NF), jnp.float32),
    mesh=_sc_mesh,
    scratch_types=[
        pltpu.VMEM((_NCHH, _CH), jnp.int32),
        pltpu.VMEM((_NCHH, _CH), jnp.int32),
        pltpu.VMEM((2, _CH, _NF), jnp.float32),
        pltpu.VMEM((2, _CH, _NF), jnp.float32),
        pltpu.SemaphoreType.DMA,
        pltpu.SemaphoreType.DMA,
        pltpu.SemaphoreType.DMA,
        pltpu.SemaphoreType.DMA,
    ],
)
def _edge_bias(ar_hbm, as_hbm, gr_hbm, gs_hbm, out_hbm,
               idx_r, idx_s, rows_r, rows_s, semr0, semr1, sems0, sems1):
    wid = lax.axis_index("s") * _NC + lax.axis_index("c")
    pltpu.sync_copy(gr_hbm.at[wid], idx_r)
    pltpu.sync_copy(gs_hbm.at[wid], idx_s)
    semr, sems = (semr0, semr1), (sems0, sems1)

    def start(c):
        p = c % 2
        return (pltpu.async_copy(ar_hbm.at[idx_r.at[c]], rows_r.at[p], semr[p]),
                pltpu.async_copy(as_hbm.at[idx_s.at[c]], rows_s.at[p], sems[p]))

    pend = start(0)
    for c in range(_NCHH):
        p = c % 2
        cr, cs = pend
        if c + 1 < _NCHH:
            pend = start(c + 1)
        cr.wait()
        cs.wait()

        def row(i, carry):
            for j in range(_NF // _L):
                sl = pl.ds(j * _L, _L)
                rows_r[p, i, sl] = jnp.maximum(
                    rows_r[p, i, sl] + rows_s[p, i, sl], 0.0)
            return carry

        lax.fori_loop(0, _CH, row, 0)
        pltpu.sync_copy(rows_r.at[p],
                        out_hbm.at[pl.ds(wid * _EWH + c * _CH, _CH)])


@functools.partial(
    pl.kernel,
    out_type=jax.ShapeDtypeStruct((_NC, _BN, _NF), jnp.float32),
    mesh=_sc_mesh,
    scratch_types=[
        pltpu.VMEM((_NCH, _CH), jnp.int32),
        pltpu.VMEM((_NCH, _CH), jnp.int32),
        pltpu.VMEM((2, _CH, _NF), jnp.float32),
        pltpu.VMEM((2, _CH, _NF), jnp.float32),
        pltpu.VMEM((_CH, _NF), jnp.float32),
        pltpu.VMEM_SHARED((_BN, _NF), jnp.float32),
        pltpu.SemaphoreType.DMA,
        pltpu.SemaphoreType.DMA,
        pltpu.SemaphoreType.DMA,
        pltpu.SemaphoreType.DMA,
        pltpu.SemaphoreType.DMA,
    ],
)
def _edge_pass(er_hbm, es_hbm, gra_hbm, grb_hbm, gsa_hbm, gsb_hbm,
               biasa_hbm, biasb_hbm, out_hbm,
               idx_r, idx_s, rows_r, rows_s, rows_b, accum,
               semr0, semr1, sems0, sems1, semb):
    cid = lax.axis_index("c")
    sid = lax.axis_index("s")
    wid = sid * _NC + cid
    semr, sems = (semr0, semr1), (sems0, sems1)

    def zrow(i, carry):
        for j in range(_NF // _L):
            rows_b[i, pl.ds(j * _L, _L)] = jnp.zeros((_L,), jnp.float32)
        return carry

    lax.fori_loop(0, _CH, zrow, 0)
    pltpu.sync_copy(rows_b, accum.at[pl.ds(sid * _RPT, _CH)])
    pltpu.sync_copy(rows_b, accum.at[pl.ds(sid * _RPT + _CH, _CH)])
    pltpu.sync_copy(gra_hbm.at[wid], idx_r.at[pl.ds(0, _NCHH)])
    pltpu.sync_copy(grb_hbm.at[wid], idx_r.at[pl.ds(_NCHH, _NCHH)])
    pltpu.sync_copy(gsa_hbm.at[wid], idx_s.at[pl.ds(0, _NCHH)])
    pltpu.sync_copy(gsb_hbm.at[wid], idx_s.at[pl.ds(_NCHH, _NCHH)])

    def start(c):
        p = c % 2
        return (pltpu.async_copy(er_hbm.at[idx_r.at[c]], rows_r.at[p], semr[p]),
                pltpu.async_copy(es_hbm.at[idx_s.at[c]], rows_s.at[p], sems[p]))

    def start_bias(c):
        if c < _NCHH:
            src_b = biasa_hbm.at[pl.ds(wid * _EWH + c * _CH, _CH)]
        else:
            src_b = biasb_hbm.at[pl.ds(wid * _EWH + (c - _NCHH) * _CH, _CH)]
        return pltpu.async_copy(src_b, rows_b, semb)

    pend = start(0)
    pend_b = start_bias(0)
    plsc.subcore_barrier()

    for c in range(_NCH):
        p = c % 2
        cr, cs = pend
        if c + 1 < _NCH:
            pend = start(c + 1)
        cr.wait()
        cs.wait()
        pend_b.wait()

        def row(i, carry):
            for j in range(_NF // _L):
                sl = pl.ds(j * _L, _L)
                rows_r[p, i, sl] = jnp.maximum(
                    rows_b[i, sl] + rows_r[p, i, sl] + rows_s[p, i, sl], 0.0)
            return carry

        lax.fori_loop(0, _CH, row, 0)
        if c + 1 < _NCH:
            pend_b = start_bias(c + 1)
        pltpu.sync_copy(rows_r.at[p], accum.at[idx_r.at[c]], add=True)

    plsc.subcore_barrier()
    for q in range(2):
        sl = pl.ds(sid * _RPT + q * _CH, _CH)
        pltpu.sync_copy(accum.at[sl], rows_s.at[q])
        pltpu.sync_copy(rows_s.at[q], out_hbm.at[cid, sl])


def _step_body(parts_ref, effect_ref, pbias_ref, pp_w1_ref, rp_w1_ref, rp_w2_ref,
               eff_ref, er_ref, es_ref):
    agg = parts_ref[0] + parts_ref[1]
    eff = jnp.maximum(
        pbias_ref[...] + jnp.dot(agg, pp_w1_ref[...]) + effect_ref[...], 0.0)
    eff_ref[...] = eff
    er_ref[...] = jnp.dot(eff, rp_w1_ref[...])
    es_ref[...] = jnp.dot(eff, rp_w2_ref[...])


_step_call = pl.pallas_call(
    _step_body,
    out_shape=[jax.ShapeDtypeStruct((_BN, _NF), jnp.float32)] * 3,
)


def _final_body(parts_ref, effect_ref, pbias_ref, pp_w1_ref, pr_w0_ref,
                pr_b0_ref, pr_w1_ref, pr_b1_ref, state3_ref, out_ref):
    agg = parts_ref[0] + parts_ref[1]
    eff = jnp.maximum(
        pbias_ref[...] + jnp.dot(agg, pp_w1_ref[...]) + effect_ref[...], 0.0)
    hh = jnp.maximum(jnp.dot(eff, pr_w0_ref[...]) + pr_b0_ref[...], 0.0)
    pred = jnp.dot(hh, pr_w1_ref[...]) + pr_b1_ref[...]
    out_ref[...] = state3_ref[...] + pred


_final_call = pl.pallas_call(
    _final_body,
    out_shape=jax.ShapeDtypeStruct((_BN, _SD), jnp.float32),
)


def kernel(attr, state, Rr, Rs, pe_w0, pe_b0, pe_w1, pe_b1, re_w0, re_b0,
           re_w1, re_b1, re_w2, re_b2, rp_w, rp_b, pp_w, pp_b, pr_w0, pr_b0,
           pr_w1, pr_b1):
    p_in = jnp.concatenate([attr, state], axis=-1).reshape(_BN, _PIN)
    state3 = state[..., :_SD].reshape(_BN, _SD)

    re_w0a, re_w0b = re_w0[:_PIN], re_w0[_PIN:]
    rp_w0, rp_w1_, rp_w2_ = rp_w[:_NF], rp_w[_NF:2 * _NF], rp_w[2 * _NF:]
    pp_w0, pp_w1 = pp_w[:_NF], pp_w[_NF:]
    r2 = lambda v: v.reshape(1, -1)

    enc_p, a_r, a_s, p_bias, er, es = _pre_call(
        p_in, pe_w0, r2(pe_b0), pe_w1, r2(pe_b1),
        re_w0a, re_w0b, r2(re_b0), rp_w1_, rp_w2_, pp_w0, r2(pp_b))

    gra_o, gsa_o = _idx_a(Rr, Rs)
    grb_o, gsb_o = _idx_b(Rr, Rs)
    gr3a = gra_o.reshape(_NW, _NCHH, _CH)
    gs3a = gsa_o.reshape(_NW, _NCHH, _CH)
    gr3b = grb_o.reshape(_NW, _NCHH, _CH)
    gs3b = gsb_o.reshape(_NW, _NCHH, _CH)

    h0a = _edge_bias(a_r, a_s, gr3a, gs3a)
    h0b = _edge_bias(a_r, a_s, gr3b, gs3b)
    rba = _relmlp_call(h0a, re_w1, r2(re_b1), re_w2, r2(re_b2),
                       rp_w0, r2(rp_b))
    rbb = _relmlp_call(h0b, re_w1, r2(re_b1), re_w2, r2(re_b2),
                       rp_w0, r2(rp_b))

    effect = enc_p
    for t in range(3):
        parts = _edge_pass(er, es, gr3a, gr3b, gs3a, gs3b, rba, rbb)
        if t < 2:
            effect, er, es = _step_call(parts, effect, p_bias, pp_w1,
                                        rp_w1_, rp_w2_)
        else:
            out = _final_call(parts, effect, p_bias, pp_w1,
                              pr_w0, r2(pr_b0), pr_w1, r2(pr_b1), state3)
    return out.reshape(_B, _N, _SD)


# speedup vs baseline: 1.0106x; 1.0106x over previous
import functools

import jax
import jax.numpy as jnp
from jax import lax
from jax.experimental import pallas as pl
from jax.experimental.pallas import tpu as pltpu
from jax.experimental.pallas import tpu_sc as plsc

_B, _N, _NREL, _NF = 4, 1024, 8192, 128
_BN, _E = _B * _N, _B * _NREL
_PIN = 14
_SD = 3

_NC, _NS, _L = 2, 16, 16
_NW = _NC * _NS
_EW = _E // _NW
_CH = 128
_NCH = _EW // _CH
_RPT = _BN // _NS


_IDXBLK = 1024
_IDXNB = _NREL // _IDXBLK


def _idx_body(rr_ref, rs_ref, gr_ref, gs_ref):
    b = pl.program_id(0)
    iota = lax.broadcasted_iota(jnp.int32, (1, _N), 1).astype(jnp.float32)
    base = b * _N
    gr_ref[0, 0, :] = jnp.sum(rr_ref[0] * iota, axis=1).astype(jnp.int32) + base
    gs_ref[0, 0, :] = jnp.sum(rs_ref[0] * iota, axis=1).astype(jnp.int32) + base


_idx_call = pl.pallas_call(
    _idx_body,
    grid=(_B, _IDXNB),
    in_specs=[
        pl.BlockSpec((1, _IDXBLK, _N), lambda b, j: (b, j, 0)),
        pl.BlockSpec((1, _IDXBLK, _N), lambda b, j: (b, j, 0)),
    ],
    out_specs=[
        pl.BlockSpec((1, 1, _IDXBLK), lambda b, j: (b * _IDXNB + j, 0, 0)),
        pl.BlockSpec((1, 1, _IDXBLK), lambda b, j: (b * _IDXNB + j, 0, 0)),
    ],
    out_shape=[
        jax.ShapeDtypeStruct((_B * _IDXNB, 1, _IDXBLK), jnp.int32),
        jax.ShapeDtypeStruct((_B * _IDXNB, 1, _IDXBLK), jnp.int32),
    ],
)


_bf = jnp.bfloat16


def _dbf(a, w):
    return jnp.dot(a.astype(_bf), w.astype(_bf),
                   preferred_element_type=jnp.float32)


def _pre_body(p_in_ref, pe_w0_ref, pe_b0_ref, pe_w1_ref, pe_b1_ref,
              re_w0a_ref, re_w0b_ref, re_b0_ref, rp_w1_ref, rp_w2_ref,
              pp_w0_ref, pp_b_ref,
              enc_p_ref, ar_ref, as_ref, pbias_ref, er_ref, es_ref):
    p_in = p_in_ref[...]
    h = jnp.maximum(_dbf(p_in, pe_w0_ref[...]) + pe_b0_ref[...], 0.0)
    enc_p = jnp.maximum(_dbf(h, pe_w1_ref[...]) + pe_b1_ref[...], 0.0)
    enc_p_ref[...] = enc_p
    ar_ref[...] = _dbf(p_in, re_w0a_ref[...]) + re_b0_ref[...]
    as_ref[...] = _dbf(p_in, re_w0b_ref[...])
    pbias_ref[...] = _dbf(enc_p, pp_w0_ref[...]) + pp_b_ref[...]
    er_ref[...] = _dbf(enc_p, rp_w1_ref[...])
    es_ref[...] = _dbf(enc_p, rp_w2_ref[...])


_pre_call = pl.pallas_call(
    _pre_body,
    out_shape=[jax.ShapeDtypeStruct((_BN, _NF), jnp.float32)] * 6,
)


_RBLK = 4096


def _relmlp_body(h0_ref, w1_ref, b1_ref, w2_ref, b2_ref, w3_ref, b3_ref, out_ref):
    h1 = jnp.maximum(_dbf(h0_ref[...], w1_ref[...]) + b1_ref[...], 0.0)
    h2 = jnp.maximum(_dbf(h1, w2_ref[...]) + b2_ref[...], 0.0)
    out_ref[...] = _dbf(h2, w3_ref[...]) + b3_ref[...]


_w_spec = pl.BlockSpec((_NF, _NF), lambda i: (0, 0))
_b_spec = pl.BlockSpec((1, _NF), lambda i: (0, 0))

_relmlp_call = pl.pallas_call(
    _relmlp_body,
    grid=(_E // _RBLK,),
    in_specs=[pl.BlockSpec((_RBLK, _NF), lambda i: (i, 0)),
              _w_spec, _b_spec, _w_spec, _b_spec, _w_spec, _b_spec],
    out_specs=pl.BlockSpec((_RBLK, _NF), lambda i: (i, 0)),
    out_shape=jax.ShapeDtypeStruct((_E, _NF), jnp.float32),
)


_sc_mesh = plsc.VectorSubcoreMesh(core_axis_name="c", subcore_axis_name="s")


@functools.partial(
    pl.kernel,
    out_type=jax.ShapeDtypeStruct((_E, _NF), jnp.float32),
    mesh=_sc_mesh,
    scratch_types=[
        pltpu.VMEM((_NCH, _CH), jnp.int32),
        pltpu.VMEM((_NCH, _CH), jnp.int32),
        pltpu.VMEM((2, _CH, _NF), jnp.float32),
        pltpu.VMEM((2, _CH, _NF), jnp.float32),
        pltpu.SemaphoreType.DMA,
        pltpu.SemaphoreType.DMA,
        pltpu.SemaphoreType.DMA,
        pltpu.SemaphoreType.DMA,
    ],
)
def _edge_bias(ar_hbm, as_hbm, gr_hbm, gs_hbm, out_hbm,
               idx_r, idx_s, rows_r, rows_s, semr0, semr1, sems0, sems1):
    wid = lax.axis_index("s") * _NC + lax.axis_index("c")
    pltpu.sync_copy(gr_hbm.at[wid], idx_r)
    pltpu.sync_copy(gs_hbm.at[wid], idx_s)
    semr, sems = (semr0, semr1), (sems0, sems1)

    def start(c):
        p = c % 2
        return (pltpu.async_copy(ar_hbm.at[idx_r.at[c]], rows_r.at[p], semr[p]),
                pltpu.async_copy(as_hbm.at[idx_s.at[c]], rows_s.at[p], sems[p]))

    pend = start(0)
    for c in range(_NCH):
        p = c % 2
        cr, cs = pend
        if c + 1 < _NCH:
            pend = start(c + 1)
        cr.wait()
        cs.wait()

        def row(i, carry):
            for j in range(_NF // _L):
                sl = pl.ds(j * _L, _L)
                rows_r[p, i, sl] = jnp.maximum(
                    rows_r[p, i, sl] + rows_s[p, i, sl], 0.0)
            return carry

        lax.fori_loop(0, _CH, row, 0)
        pltpu.sync_copy(rows_r.at[p], out_hbm.at[pl.ds(wid * _EW + c * _CH, _CH)])


@functools.partial(
    pl.kernel,
    out_type=jax.ShapeDtypeStruct((_NC, _BN, _NF), jnp.float32),
    mesh=_sc_mesh,
    scratch_types=[
        pltpu.VMEM((_NCH, _CH), jnp.int32),
        pltpu.VMEM((_NCH, _CH), jnp.int32),
        pltpu.VMEM((2, _CH, _NF), jnp.float32),
        pltpu.VMEM((2, _CH, _NF), jnp.float32),
        pltpu.VMEM((_CH, _NF), jnp.float32),
        pltpu.VMEM_SHARED((_BN, _NF), jnp.float32),
        pltpu.SemaphoreType.DMA,
        pltpu.SemaphoreType.DMA,
        pltpu.SemaphoreType.DMA,
        pltpu.SemaphoreType.DMA,
        pltpu.SemaphoreType.DMA,
    ],
)
def _edge_pass(er_hbm, es_hbm, gr_hbm, gs_hbm, bias_hbm, out_hbm,
               idx_r, idx_s, rows_r, rows_s, rows_b, accum,
               semr0, semr1, sems0, sems1, semb):
    cid = lax.axis_index("c")
    sid = lax.axis_index("s")
    wid = sid * _NC + cid
    semr, sems = (semr0, semr1), (sems0, sems1)

    def zrow(i, carry):
        for j in range(_NF // _L):
            rows_b[i, pl.ds(j * _L, _L)] = jnp.zeros((_L,), jnp.float32)
        return carry

    lax.fori_loop(0, _CH, zrow, 0)
    pltpu.sync_copy(rows_b, accum.at[pl.ds(sid * _RPT, _CH)])
    pltpu.sync_copy(rows_b, accum.at[pl.ds(sid * _RPT + _CH, _CH)])
    pltpu.sync_copy(gr_hbm.at[wid], idx_r)
    pltpu.sync_copy(gs_hbm.at[wid], idx_s)

    def start(c):
        p = c % 2
        return (pltpu.async_copy(er_hbm.at[idx_r.at[c]], rows_r.at[p], semr[p]),
                pltpu.async_copy(es_hbm.at[idx_s.at[c]], rows_s.at[p], sems[p]))

    def start_bias(c):
        return pltpu.async_copy(bias_hbm.at[pl.ds(wid * _EW + c * _CH, _CH)],
                                rows_b, semb)

    pend = start(0)
    pend_b = start_bias(0)
    plsc.subcore_barrier()

    for c in range(_NCH):
        p = c % 2
        cr, cs = pend
        if c + 1 < _NCH:
            pend = start(c + 1)
        cr.wait()
        cs.wait()
        pend_b.wait()

        def row(i, carry):
            for j in range(_NF // _L):
                sl = pl.ds(j * _L, _L)
                v = jnp.maximum(
                    rows_b[i, sl] + rows_r[p, i, sl] + rows_s[p, i, sl], 0.0)
                x = lax.bitcast_convert_type(v, jnp.int32)
                x = (x + ((x >> 16) & 1) + 32767) & (-65536)
                rows_r[p, i, sl] = lax.bitcast_convert_type(x, jnp.float32)
            return carry

        lax.fori_loop(0, _CH, row, 0)
        if c + 1 < _NCH:
            pend_b = start_bias(c + 1)
        pltpu.sync_copy(rows_r.at[p], accum.at[idx_r.at[c]], add=True)

    plsc.subcore_barrier()
    for q in range(2):
        sl = pl.ds(sid * _RPT + q * _CH, _CH)
        pltpu.sync_copy(accum.at[sl], rows_s.at[q])
        pltpu.sync_copy(rows_s.at[q], out_hbm.at[cid, sl])


def _step_body(parts_ref, effect_ref, pbias_ref, pp_w1_ref, rp_w1_ref, rp_w2_ref,
               eff_ref, er_ref, es_ref):
    agg = parts_ref[0] + parts_ref[1]
    eff = jnp.maximum(
        pbias_ref[...] + _dbf(agg, pp_w1_ref[...]) + effect_ref[...], 0.0)
    eff_ref[...] = eff
    er_ref[...] = _dbf(eff, rp_w1_ref[...])
    es_ref[...] = _dbf(eff, rp_w2_ref[...])


_step_call = pl.pallas_call(
    _step_body,
    out_shape=[jax.ShapeDtypeStruct((_BN, _NF), jnp.float32)] * 3,
)


def _final_body(parts_ref, effect_ref, pbias_ref, pp_w1_ref, pr_w0_ref,
                pr_b0_ref, pr_w1_ref, pr_b1_ref, state3_ref, out_ref):
    agg = parts_ref[0] + parts_ref[1]
    eff = jnp.maximum(
        pbias_ref[...] + _dbf(agg, pp_w1_ref[...]) + effect_ref[...], 0.0)
    hh = jnp.maximum(_dbf(eff, pr_w0_ref[...]) + pr_b0_ref[...], 0.0)
    pred = _dbf(hh, pr_w1_ref[...]) + pr_b1_ref[...]
    out_ref[...] = state3_ref[...] + pred


_final_call = pl.pallas_call(
    _final_body,
    out_shape=jax.ShapeDtypeStruct((_BN, _SD), jnp.float32),
)


def kernel(attr, state, Rr, Rs, pe_w0, pe_b0, pe_w1, pe_b1, re_w0, re_b0,
           re_w1, re_b1, re_w2, re_b2, rp_w, rp_b, pp_w, pp_b, pr_w0, pr_b0,
           pr_w1, pr_b1):
    p_in = jnp.concatenate([attr, state], axis=-1).reshape(_BN, _PIN)
    state3 = state[..., :_SD].reshape(_BN, _SD)

    re_w0a, re_w0b = re_w0[:_PIN], re_w0[_PIN:]
    rp_w0, rp_w1_, rp_w2_ = rp_w[:_NF], rp_w[_NF:2 * _NF], rp_w[2 * _NF:]
    pp_w0, pp_w1 = pp_w[:_NF], pp_w[_NF:]
    r2 = lambda v: v.reshape(1, -1)

    gr_o, gs_o = _idx_call(Rr, Rs)
    gr3 = gr_o.reshape(_NW, _NCH, _CH)
    gs3 = gs_o.reshape(_NW, _NCH, _CH)

    enc_p, a_r, a_s, p_bias, er, es = _pre_call(
        p_in, pe_w0, r2(pe_b0), pe_w1, r2(pe_b1),
        re_w0a, re_w0b, r2(re_b0), rp_w1_, rp_w2_, pp_w0, r2(pp_b))

    h0 = _edge_bias(a_r, a_s, gr3, gs3)
    rel_bias = _relmlp_call(h0, re_w1, r2(re_b1), re_w2, r2(re_b2),
                            rp_w0, r2(rp_b))

    effect = enc_p
    for t in range(3):
        parts = _edge_pass(er, es, gr3, gs3, rel_bias)
        if t < 2:
            effect, er, es = _step_call(parts, effect, p_bias, pp_w1,
                                        rp_w1_, rp_w2_)
        else:
            out = _final_call(parts, effect, p_bias, pp_w1,
                              pr_w0, r2(pr_b0), pr_w1, r2(pr_b1), state3)
    return out.reshape(_B, _N, _SD)
